# baseline (speedup 1.0000x reference)
import jax
import jax.numpy as jnp
from jax import lax
from jax.experimental import pallas as pl
from jax.experimental.pallas import tpu as pltpu
from jax.experimental.pallas import tpu_sc as plsc

NUM_CORES = 2
NUM_SUBCORES = 16
LANES = 16
NUM_WORKERS = NUM_CORES * NUM_SUBCORES

BATCH = 16384
FACTORS = 32
ROWS128 = 250000
B_PER_W = BATCH // NUM_WORKERS
HALF = B_PER_W // 2


def _sc_body(user_hbm, item_hbm, utab_hbm, itab_hbm, out_hbm,
             uidx_v, iidx_v, urow_v, irow_v, urows_v, irows_v, out_v,
             usem, isem):
    wid = lax.axis_index("s") * NUM_CORES + lax.axis_index("c")
    base = wid * B_PER_W

    pltpu.sync_copy(user_hbm.at[pl.ds(base, B_PER_W)], uidx_v)
    pltpu.sync_copy(item_hbm.at[pl.ds(base, B_PER_W)], iidx_v)

    def quarter(blk, carry):
        uvec = uidx_v[pl.ds(blk * LANES, LANES)]
        ivec = iidx_v[pl.ds(blk * LANES, LANES)]
        urow_v[pl.ds(blk * LANES, LANES)] = uvec >> 2
        irow_v[pl.ds(blk * LANES, LANES)] = ivec >> 2
        return carry

    lax.fori_loop(0, B_PER_W // LANES, quarter, 0)

    lane = lax.iota(jnp.int32, LANES)

    for h in range(2):
        ucopy = pltpu.async_copy(
            utab_hbm.at[urow_v.at[pl.ds(h * HALF, HALF)]], urows_v, usem)
        icopy = pltpu.async_copy(
            itab_hbm.at[irow_v.at[pl.ds(h * HALF, HALF)]], irows_v, isem)
        ucopy.wait()
        icopy.wait()

        def group(g, carry):
            rows = g * LANES + lane
            uvec = uidx_v[pl.ds(h * HALF + g * LANES, LANES)]
            ivec = iidx_v[pl.ds(h * HALF + g * LANES, LANES)]
            ucol0 = (uvec & 3) * FACTORS
            icol0 = (ivec & 3) * FACTORS
            acc = jnp.zeros((LANES,), jnp.float32)
            for f in range(FACTORS):
                uval = plsc.load_gather(urows_v, [rows, ucol0 + f])
                ival = plsc.load_gather(irows_v, [rows, icol0 + f])
                acc = acc + uval * ival
            out_v[pl.ds(h * HALF + g * LANES, LANES)] = acc
            return carry

        lax.fori_loop(0, HALF // LANES, group, 0)

    pltpu.sync_copy(out_v, out_hbm.at[pl.ds(base, B_PER_W)])


@jax.jit
def kernel(user, item, user_table, item_table):
    call = pl.kernel(
        _sc_body,
        out_type=jax.ShapeDtypeStruct((BATCH,), jnp.float32),
        mesh=plsc.VectorSubcoreMesh(
            core_axis_name="c", subcore_axis_name="s",
            num_cores=NUM_CORES, num_subcores=NUM_SUBCORES),
        compiler_params=pltpu.CompilerParams(
            needs_layout_passes=False, use_tc_tiling_on_sc=False),
        scratch_types=[
            pltpu.VMEM((B_PER_W,), jnp.int32),
            pltpu.VMEM((B_PER_W,), jnp.int32),
            pltpu.VMEM((B_PER_W,), jnp.int32),
            pltpu.VMEM((B_PER_W,), jnp.int32),
            pltpu.VMEM((HALF, 128), jnp.float32),
            pltpu.VMEM((HALF, 128), jnp.float32),
            pltpu.VMEM((B_PER_W,), jnp.float32),
            pltpu.SemaphoreType.DMA,
            pltpu.SemaphoreType.DMA,
        ],
    )
    return call(user.astype(jnp.int32), item.astype(jnp.int32),
                user_table.reshape(ROWS128, 128),
                item_table.reshape(ROWS128, 128))


# speedup vs baseline: 3.5811x; 3.5811x over previous
import jax
import jax.numpy as jnp
from jax import lax
from jax.experimental import pallas as pl
from jax.experimental.pallas import tpu as pltpu
from jax.experimental.pallas import tpu_sc as plsc

NUM_CORES = 2
NUM_SUBCORES = 16
LANES = 16
NUM_WORKERS = NUM_CORES * NUM_SUBCORES

BATCH = 16384
FACTORS = 32
B_PER_W = BATCH // NUM_WORKERS
GROUP = 8
N_GROUPS = B_PER_W // GROUP
PAD = LANES


def _sc_body(user_hbm, item_hbm, utabT, itabT, out_hbm,
             uidx_v, iidx_v, ustage, istage, out_v, usem, isem):
    wid = lax.axis_index("s") * NUM_CORES + lax.axis_index("c")
    base = wid * B_PER_W

    pltpu.sync_copy(user_hbm.at[pl.ds(base, B_PER_W)], uidx_v.at[pl.ds(0, B_PER_W)])
    pltpu.sync_copy(item_hbm.at[pl.ds(base, B_PER_W)], iidx_v.at[pl.ds(0, B_PER_W)])

    lane = lax.iota(jnp.int32, LANES)
    slotv = lane & (GROUP - 1)
    lomask = lane < GROUP

    def group(g, carry):
        uvec = uidx_v[pl.ds(g * GROUP, LANES)]
        ivec = iidx_v[pl.ds(g * GROUP, LANES)]
        handles = []
        for j in range(GROUP):
            ucs = pl.multiple_of((uvec[j] >> 7) * 128, 128)
            ics = pl.multiple_of((ivec[j] >> 7) * 128, 128)
            handles.append(pltpu.async_copy(
                utabT.at[:, pl.ds(ucs, 128)], ustage.at[j], usem))
            handles.append(pltpu.async_copy(
                itabT.at[:, pl.ds(ics, 128)], istage.at[j], isem))
        for h in handles:
            h.wait()

        ucol = uvec & 127
        icol = ivec & 127
        acc = jnp.zeros((LANES,), jnp.float32)
        for f in range(FACTORS):
            fv = jnp.full((LANES,), f, jnp.int32)
            uval = plsc.load_gather(ustage, [slotv, fv, ucol])
            ival = plsc.load_gather(istage, [slotv, fv, icol])
            acc = acc + uval * ival
        plsc.store_compressed(out_v.at[pl.ds(g * GROUP, LANES)], acc,
                              mask=lomask)
        return carry

    lax.fori_loop(0, N_GROUPS, group, 0)

    pltpu.sync_copy(out_v.at[pl.ds(0, B_PER_W)],
                    out_hbm.at[pl.ds(base, B_PER_W)])


@jax.jit
def kernel(user, item, user_table, item_table):
    call = pl.kernel(
        _sc_body,
        out_type=jax.ShapeDtypeStruct((BATCH,), jnp.float32),
        mesh=plsc.VectorSubcoreMesh(
            core_axis_name="c", subcore_axis_name="s",
            num_cores=NUM_CORES, num_subcores=NUM_SUBCORES),
        compiler_params=pltpu.CompilerParams(
            needs_layout_passes=False, use_tc_tiling_on_sc=True),
        scratch_types=[
            pltpu.VMEM((B_PER_W + PAD,), jnp.int32),
            pltpu.VMEM((B_PER_W + PAD,), jnp.int32),
            pltpu.VMEM((GROUP, FACTORS, 128), jnp.float32),
            pltpu.VMEM((GROUP, FACTORS, 128), jnp.float32),
            pltpu.VMEM((B_PER_W + PAD,), jnp.float32),
            pltpu.SemaphoreType.DMA,
            pltpu.SemaphoreType.DMA,
        ],
    )
    return call(user.astype(jnp.int32), item.astype(jnp.int32),
                user_table.T, item_table.T)
